# baseline (speedup 1.0000x reference)
import functools

import jax
import jax.numpy as jnp
from jax import lax
from jax.experimental import pallas as pl
from jax.experimental.pallas import tpu as pltpu
from jax.experimental.pallas import tpu_sc as plsc

_HIDDEN = 128
_NVREG = _HIDDEN // 16
_L = 50
_STRIDE = 56
_CROWS = 64
_NBUF = 8
_GROUP = 16


@functools.partial(jax.jit, static_argnums=(4, 5))
def _embed(idx, tf, vocab_table, type_table, n_batch, n_workers):
    b_per_w = n_batch // n_workers
    n_chunks = b_per_w
    n_outer = n_chunks // _NBUF
    per_w = n_chunks * _STRIDE
    nc = plsc.get_sparse_core_info().num_cores

    def body(idx_hbm, tf_hbm, vt_hbm, tt_hbm, out_hbm, *refs):
        idx_v, tf_v, tt_v = refs[0], refs[1], refs[2]
        rows = refs[3:3 + _NBUF]
        gsem = refs[3 + _NBUF:3 + 2 * _NBUF]
        wsem = refs[3 + 2 * _NBUF:3 + 3 * _NBUF]

        wid = lax.axis_index("s") * nc + lax.axis_index("c")
        base = wid * per_w

        pltpu.sync_copy(idx_hbm.at[pl.ds(base, per_w)], idx_v.at[pl.ds(0, per_w)])
        pltpu.sync_copy(tf_hbm.at[pl.ds(base, per_w)], tf_v.at[pl.ds(0, per_w)])
        pltpu.sync_copy(tt_hbm, tt_v)
        t0 = [tt_v[0, pl.ds(16 * k, 16)] for k in range(_NVREG)]
        dt = [tt_v[1, pl.ds(16 * k, 16)] - t0[k] for k in range(_NVREG)]

        def gather_args(c, b):
            return (vt_hbm.at[idx_v.at[pl.ds(c * _STRIDE, _STRIDE)]],
                    rows[b].at[pl.ds(0, _STRIDE)], gsem[b])

        def writeback_args(c, b):
            return (rows[b].at[pl.ds(0, _L)],
                    out_hbm.at[wid * b_per_w + c], wsem[b])

        def gather(c, b):
            pltpu.async_copy(*gather_args(c, b))

        def gather_wait(c, b):
            pltpu.make_async_copy(*gather_args(c, b)).wait()

        def writeback(c, b):
            pltpu.async_copy(*writeback_args(c, b))

        def writeback_wait(c, b):
            pltpu.make_async_copy(*writeback_args(c, b)).wait()

        for c in range(_NBUF - 2):
            gather(c, c)

        def compute(buf, c):
            def group_body(g, carry):
                tvec = tf_v[pl.ds(c * _STRIDE + g * _GROUP, _GROUP)]
                for j in range(_GROUP):
                    tsp = tvec.at[jnp.full((16,), j, jnp.int32)].get(
                        mode="promise_in_bounds")
                    r = g * _GROUP + j
                    for k in range(_NVREG):
                        sl = pl.ds(16 * k, 16)
                        buf[r, sl] = buf[r, sl] + (t0[k] + tsp * dt[k])
                return carry

            lax.fori_loop(0, _CROWS // _GROUP, group_body, 0)

        def outer_body(gi, carry):
            for b in range(_NBUF):
                c = gi * _NBUF + b
                gather_wait(c, b)
                compute(rows[b], c)
                writeback(c, b)
                br = (b + _NBUF - 2) % _NBUF

                @pl.when(c >= 2)
                def _():
                    writeback_wait(c - 2, br)

                @pl.when(c <= n_chunks - 1 - (_NBUF - 2))
                def _():
                    gather(c + _NBUF - 2, br)

            return carry

        lax.fori_loop(0, n_outer, outer_body, 0)
        writeback_wait(n_chunks - 2, (n_chunks - 2) % _NBUF)
        writeback_wait(n_chunks - 1, (n_chunks - 1) % _NBUF)

    return pl.kernel(
        body,
        out_type=jax.ShapeDtypeStruct((n_batch, _L, _HIDDEN), jnp.float32),
        mesh=plsc.VectorSubcoreMesh(core_axis_name="c", subcore_axis_name="s"),
        compiler_params=pltpu.CompilerParams(needs_layout_passes=False),
        scratch_types=(
            [
                pltpu.VMEM((per_w + _GROUP,), jnp.int32),
                pltpu.VMEM((per_w + _GROUP,), jnp.float32),
                pltpu.VMEM((2, _HIDDEN), jnp.float32),
            ]
            + [pltpu.VMEM((_CROWS, _HIDDEN), jnp.float32)] * _NBUF
            + [pltpu.SemaphoreType.DMA] * (2 * _NBUF)
        ),
    )(idx, tf, vocab_table, type_table)


def kernel(vocab, type, vocab_table, type_table):
    b, l = vocab.shape
    info = plsc.get_sparse_core_info()
    n_workers = info.num_cores * info.num_subcores
    npad = _STRIDE - l
    filler = jnp.arange(b * npad, dtype=jnp.int32).reshape(b, npad) % 4096
    idx = jnp.concatenate([vocab, filler], axis=1).reshape(-1)
    tf = jnp.concatenate(
        [type.astype(jnp.float32), jnp.zeros((b, npad), jnp.float32)],
        axis=1).reshape(-1)
    return _embed(idx, tf, vocab_table, type_table, b, n_workers)


# speedup vs baseline: 1.1114x; 1.1114x over previous
import functools

import jax
import jax.numpy as jnp
from jax import lax
from jax.experimental import pallas as pl
from jax.experimental.pallas import tpu as pltpu
from jax.experimental.pallas import tpu_sc as plsc

_HIDDEN = 128
_NVREG = _HIDDEN // 16
_L = 50
_BPC = 2
_STRIDE = 104
_CROWS = 112
_NBUF = 8
_GROUP = 16


@functools.partial(jax.jit, static_argnums=(4, 5))
def _embed(idx, tf, vocab_table, type_table, n_batch, n_workers):
    b_per_w = n_batch // n_workers
    n_chunks = b_per_w // _BPC
    n_outer = n_chunks // _NBUF
    per_w = n_chunks * _STRIDE
    nc = plsc.get_sparse_core_info().num_cores

    def body(idx_hbm, tf_hbm, vt_hbm, tt_hbm, out_hbm, *refs):
        idx_v, tf_v, tt_v = refs[0], refs[1], refs[2]
        rows = refs[3:3 + _NBUF]
        gsem = refs[3 + _NBUF:3 + 2 * _NBUF]
        wsem = refs[3 + 2 * _NBUF:3 + 3 * _NBUF]

        wid = lax.axis_index("s") * nc + lax.axis_index("c")
        base = wid * per_w

        pltpu.sync_copy(idx_hbm.at[pl.ds(base, per_w)], idx_v.at[pl.ds(0, per_w)])
        pltpu.sync_copy(tf_hbm.at[pl.ds(base, per_w)], tf_v.at[pl.ds(0, per_w)])
        pltpu.sync_copy(tt_hbm, tt_v)
        t0 = [tt_v[0, pl.ds(16 * k, 16)] for k in range(_NVREG)]
        dt = [tt_v[1, pl.ds(16 * k, 16)] - t0[k] for k in range(_NVREG)]

        def gather_args(c, b):
            return (vt_hbm.at[idx_v.at[pl.ds(c * _STRIDE, _STRIDE)]],
                    rows[b].at[pl.ds(0, _STRIDE)], gsem[b])

        def writeback_args(c, b, e):
            return (rows[b].at[pl.ds(e * _L, _L)],
                    out_hbm.at[(wid * n_chunks + c) * _BPC + e], wsem[b])

        def gather(c, b):
            pltpu.async_copy(*gather_args(c, b))

        def gather_wait(c, b):
            pltpu.make_async_copy(*gather_args(c, b)).wait()

        def writeback(c, b):
            for e in range(_BPC):
                pltpu.async_copy(*writeback_args(c, b, e))

        def writeback_wait(c, b):
            for e in range(_BPC):
                pltpu.make_async_copy(*writeback_args(c, b, e)).wait()

        for c in range(_NBUF - 2):
            gather(c, c)

        def compute(buf, c):
            def group_body(g, carry):
                tvec = tf_v[pl.ds(c * _STRIDE + g * _GROUP, _GROUP)]
                for j in range(_GROUP):
                    tsp = tvec.at[jnp.full((16,), j, jnp.int32)].get(
                        mode="promise_in_bounds")
                    r = g * _GROUP + j
                    for k in range(_NVREG):
                        sl = pl.ds(16 * k, 16)
                        buf[r, sl] = buf[r, sl] + (t0[k] + tsp * dt[k])
                return carry

            lax.fori_loop(0, _CROWS // _GROUP, group_body, 0)

        def outer_body(gi, carry):
            for b in range(_NBUF):
                c = gi * _NBUF + b
                gather_wait(c, b)
                compute(rows[b], c)
                writeback(c, b)
                br = (b + _NBUF - 2) % _NBUF

                @pl.when(c >= 2)
                def _():
                    writeback_wait(c - 2, br)

                @pl.when(c <= n_chunks - 1 - (_NBUF - 2))
                def _():
                    gather(c + _NBUF - 2, br)

            return carry

        lax.fori_loop(0, n_outer, outer_body, 0)
        writeback_wait(n_chunks - 2, (n_chunks - 2) % _NBUF)
        writeback_wait(n_chunks - 1, (n_chunks - 1) % _NBUF)

    return pl.kernel(
        body,
        out_type=jax.ShapeDtypeStruct((n_batch, _L, _HIDDEN), jnp.float32),
        mesh=plsc.VectorSubcoreMesh(core_axis_name="c", subcore_axis_name="s"),
        compiler_params=pltpu.CompilerParams(needs_layout_passes=False),
        scratch_types=(
            [
                pltpu.VMEM((per_w + _GROUP,), jnp.int32),
                pltpu.VMEM((per_w + _GROUP,), jnp.float32),
                pltpu.VMEM((2, _HIDDEN), jnp.float32),
            ]
            + [pltpu.VMEM((_CROWS, _HIDDEN), jnp.float32)] * _NBUF
            + [pltpu.SemaphoreType.DMA] * (2 * _NBUF)
        ),
    )(idx, tf, vocab_table, type_table)


def kernel(vocab, type, vocab_table, type_table):
    b, l = vocab.shape
    info = plsc.get_sparse_core_info()
    n_workers = info.num_cores * info.num_subcores
    nch = b // _BPC
    npad = _STRIDE - _BPC * l
    filler = jnp.arange(nch * npad, dtype=jnp.int32).reshape(nch, npad) % 4096
    idx = jnp.concatenate(
        [vocab.reshape(nch, _BPC * l), filler], axis=1).reshape(-1)
    tf = jnp.concatenate(
        [type.astype(jnp.float32).reshape(nch, _BPC * l),
         jnp.zeros((nch, npad), jnp.float32)], axis=1).reshape(-1)
    return _embed(idx, tf, vocab_table, type_table, b, n_workers)


# speedup vs baseline: 1.9610x; 1.7645x over previous
import functools

import jax
import jax.numpy as jnp
from jax import lax
from jax.experimental import pallas as pl
from jax.experimental.pallas import tpu as pltpu
from jax.experimental.pallas import tpu_sc as plsc

_HIDDEN = 128
_NVREG = _HIDDEN // 16
_CHUNK = 128
_NBUF = 5
_GROUP = 16
_BW = 16
_LW = 2


@functools.partial(jax.jit, static_argnums=(4, 5, 6))
def _embed(idx, tf, vocab_table, type_table, n_batch, n_l, n_workers):
    n_rows = n_batch * n_l
    per_w = n_rows // n_workers
    n_chunks = per_w // _CHUNK
    n_outer = n_chunks // _NBUF
    l_per_w = n_l // _LW
    b_per_w = n_batch // _BW
    nc = plsc.get_sparse_core_info().num_cores

    def body(idx_hbm, tf_hbm, vt_hbm, tt_hbm, out_hbm, *refs):
        idx_v, tf_v, tt_v = refs[0], refs[1], refs[2]
        rows = refs[3:3 + _NBUF]
        gsem = refs[3 + _NBUF:3 + 2 * _NBUF]
        wsem = refs[3 + 2 * _NBUF:3 + 3 * _NBUF]

        wid = lax.axis_index("s") * nc + lax.axis_index("c")
        base = wid * per_w
        lh = wid // _BW
        bb = wid % _BW

        pltpu.sync_copy(idx_hbm.at[pl.ds(base, per_w)], idx_v)
        pltpu.sync_copy(tf_hbm.at[pl.ds(base, per_w)], tf_v)
        pltpu.sync_copy(tt_hbm, tt_v)
        t0 = [tt_v[0, pl.ds(16 * k, 16)] for k in range(_NVREG)]
        dt = [tt_v[1, pl.ds(16 * k, 16)] - t0[k] for k in range(_NVREG)]

        def out_off(c):
            l = lh * l_per_w + c // 2
            b0 = bb * b_per_w + (c % 2) * _CHUNK
            return l * n_batch + b0

        def gather_args(c, b):
            return (vt_hbm.at[idx_v.at[pl.ds(c * _CHUNK, _CHUNK)]], rows[b],
                    gsem[b])

        def writeback_args(c, b):
            return (rows[b], out_hbm.at[pl.ds(out_off(c), _CHUNK)], wsem[b])

        def gather(c, b):
            pltpu.async_copy(*gather_args(c, b))

        def gather_wait(c, b):
            pltpu.make_async_copy(*gather_args(c, b)).wait()

        def writeback(c, b):
            pltpu.async_copy(*writeback_args(c, b))

        def writeback_wait(c, b):
            pltpu.make_async_copy(*writeback_args(c, b)).wait()

        for c in range(_NBUF - 2):
            gather(c, c)

        def compute(buf, c):
            def group_body(g, carry):
                tvec = tf_v[pl.ds(c * _CHUNK + g * _GROUP, _GROUP)]
                for j in range(_GROUP):
                    tsp = tvec.at[jnp.full((16,), j, jnp.int32)].get(
                        mode="promise_in_bounds")
                    r = g * _GROUP + j
                    for k in range(_NVREG):
                        sl = pl.ds(16 * k, 16)
                        buf[r, sl] = buf[r, sl] + (t0[k] + tsp * dt[k])
                return carry

            lax.fori_loop(0, _CHUNK // _GROUP, group_body, 0)

        def outer_body(gi, carry):
            for b in range(_NBUF):
                c = gi * _NBUF + b
                gather_wait(c, b)
                compute(rows[b], c)
                writeback(c, b)
                br = (b + _NBUF - 2) % _NBUF

                @pl.when(c >= 2)
                def _():
                    writeback_wait(c - 2, br)

                @pl.when(c <= n_chunks - 1 - (_NBUF - 2))
                def _():
                    gather(c + _NBUF - 2, br)

            return carry

        lax.fori_loop(0, n_outer, outer_body, 0)
        writeback_wait(n_chunks - 2, (n_chunks - 2) % _NBUF)
        writeback_wait(n_chunks - 1, (n_chunks - 1) % _NBUF)

    return pl.kernel(
        body,
        out_type=jax.ShapeDtypeStruct((n_rows, _HIDDEN), jnp.float32),
        mesh=plsc.VectorSubcoreMesh(core_axis_name="c", subcore_axis_name="s"),
        compiler_params=pltpu.CompilerParams(needs_layout_passes=False),
        scratch_types=(
            [
                pltpu.VMEM((per_w,), jnp.int32),
                pltpu.VMEM((per_w,), jnp.float32),
                pltpu.VMEM((2, _HIDDEN), jnp.float32),
            ]
            + [pltpu.VMEM((_CHUNK, _HIDDEN), jnp.float32)] * _NBUF
            + [pltpu.SemaphoreType.DMA] * (2 * _NBUF)
        ),
    )(idx, tf, vocab_table, type_table)


def _worker_order(x, b, l):
    lpw, bpw = l // _LW, b // _BW
    xt = x.T.reshape(_LW, lpw, _BW, bpw // _CHUNK, _CHUNK)
    return xt.transpose(0, 2, 1, 3, 4).reshape(-1)


def kernel(vocab, type, vocab_table, type_table):
    b, l = vocab.shape
    info = plsc.get_sparse_core_info()
    n_workers = info.num_cores * info.num_subcores
    idx = _worker_order(vocab, b, l)
    tf = _worker_order(type.astype(jnp.float32), b, l)
    out = _embed(idx, tf, vocab_table, type_table, b, l, n_workers)
    return out.reshape(l, b, _HIDDEN).transpose(1, 0, 2)


# speedup vs baseline: 1.9636x; 1.0013x over previous
import functools

import jax
import jax.numpy as jnp
from jax import lax
from jax.experimental import pallas as pl
from jax.experimental.pallas import tpu as pltpu
from jax.experimental.pallas import tpu_sc as plsc

_HIDDEN = 128
_NVREG = _HIDDEN // 16
_CHUNK = 128
_NBUF = 5
_GROUP = 16
_BW = 16
_LW = 2


@functools.partial(jax.jit, static_argnums=(4, 5, 6))
def _embed(idx, tf, vocab_table, type_table, n_batch, n_l, n_workers):
    n_rows = n_batch * n_l
    per_w = n_rows // n_workers
    n_chunks = per_w // _CHUNK
    n_outer = n_chunks // _NBUF
    l_per_w = n_l // _LW
    b_per_w = n_batch // _BW
    nc = plsc.get_sparse_core_info().num_cores

    def body(idx_hbm, tf_hbm, vt_hbm, tt_hbm, out_hbm, *refs):
        idx_v, tf_v, tt_v = refs[0], refs[1], refs[2]
        rows = refs[3:3 + _NBUF]
        gsem = refs[3 + _NBUF:3 + 2 * _NBUF]
        wsem = refs[3 + 2 * _NBUF:3 + 3 * _NBUF]

        wid = lax.axis_index("s") * nc + lax.axis_index("c")
        base = wid * per_w
        lh = wid // _BW
        bb = wid % _BW

        pltpu.sync_copy(idx_hbm.at[pl.ds(base, per_w)], idx_v)
        pltpu.sync_copy(tf_hbm.at[pl.ds(base, per_w)], tf_v)
        pltpu.sync_copy(tt_hbm, tt_v)
        t0 = [tt_v[0, pl.ds(16 * k, 16)] for k in range(_NVREG)]
        dt = [tt_v[1, pl.ds(16 * k, 16)] - t0[k] for k in range(_NVREG)]

        def out_off(c):
            l = lh * l_per_w + c // 2
            b0 = bb * b_per_w + (c % 2) * _CHUNK
            return l * n_batch + b0

        def gather_args(c, b):
            return (vt_hbm.at[idx_v.at[pl.ds(c * _CHUNK, _CHUNK)]], rows[b],
                    gsem[b])

        def writeback_args(c, b):
            return (rows[b], out_hbm.at[pl.ds(out_off(c), _CHUNK)], wsem[b])

        def gather(c, b):
            pltpu.async_copy(*gather_args(c, b))

        def gather_wait(c, b):
            pltpu.make_async_copy(*gather_args(c, b)).wait()

        def writeback(c, b):
            pltpu.async_copy(*writeback_args(c, b))

        def writeback_wait(c, b):
            pltpu.make_async_copy(*writeback_args(c, b)).wait()

        for c in range(_NBUF - 1):
            gather(c, c)

        def compute(buf, c):
            def group_body(g, carry):
                tvec = tf_v[pl.ds(c * _CHUNK + g * _GROUP, _GROUP)]
                for j in range(_GROUP):
                    tsp = tvec.at[jnp.full((16,), j, jnp.int32)].get(
                        mode="promise_in_bounds")
                    r = g * _GROUP + j
                    for k in range(_NVREG):
                        sl = pl.ds(16 * k, 16)
                        buf[r, sl] = buf[r, sl] + (t0[k] + tsp * dt[k])
                return carry

            lax.fori_loop(0, _CHUNK // _GROUP, group_body, 0)

        def outer_body(gi, carry):
            for b in range(_NBUF):
                c = gi * _NBUF + b
                gather_wait(c, b)
                compute(rows[b], c)
                writeback(c, b)
                br = (b + _NBUF - 1) % _NBUF

                @pl.when(c >= 1)
                def _():
                    writeback_wait(c - 1, br)

                @pl.when(c <= n_chunks - 1 - (_NBUF - 1))
                def _():
                    gather(c + _NBUF - 1, br)

            return carry

        lax.fori_loop(0, n_outer, outer_body, 0)
        writeback_wait(n_chunks - 1, (n_chunks - 1) % _NBUF)

    return pl.kernel(
        body,
        out_type=jax.ShapeDtypeStruct((n_rows, _HIDDEN), jnp.float32),
        mesh=plsc.VectorSubcoreMesh(core_axis_name="c", subcore_axis_name="s"),
        compiler_params=pltpu.CompilerParams(needs_layout_passes=False),
        scratch_types=(
            [
                pltpu.VMEM((per_w,), jnp.int32),
                pltpu.VMEM((per_w,), jnp.float32),
                pltpu.VMEM((2, _HIDDEN), jnp.float32),
            ]
            + [pltpu.VMEM((_CHUNK, _HIDDEN), jnp.float32)] * _NBUF
            + [pltpu.SemaphoreType.DMA] * (2 * _NBUF)
        ),
    )(idx, tf, vocab_table, type_table)


def _worker_order(x, b, l):
    lpw, bpw = l // _LW, b // _BW
    xt = x.T.reshape(_LW, lpw, _BW, bpw // _CHUNK, _CHUNK)
    return xt.transpose(0, 2, 1, 3, 4).reshape(-1)


def kernel(vocab, type, vocab_table, type_table):
    b, l = vocab.shape
    info = plsc.get_sparse_core_info()
    n_workers = info.num_cores * info.num_subcores
    idx = _worker_order(vocab, b, l)
    tf = _worker_order(type.astype(jnp.float32), b, l)
    out = _embed(idx, tf, vocab_table, type_table, b, l, n_workers)
    return out.reshape(l, b, _HIDDEN).transpose(1, 0, 2)


# speedup vs baseline: 1.9643x; 1.0004x over previous
import functools

import jax
import jax.numpy as jnp
from jax import lax
from jax.experimental import pallas as pl
from jax.experimental.pallas import tpu as pltpu
from jax.experimental.pallas import tpu_sc as plsc

_HIDDEN = 128
_NVREG = _HIDDEN // 16
_CHUNK = 128
_NBUF = 5
_GROUP = 16
_BW = 16
_LW = 2


@functools.partial(jax.jit, static_argnums=(4, 5, 6))
def _embed(idx, tf, vocab_table, type_table, n_batch, n_l, n_workers):
    n_rows = n_batch * n_l
    per_w = n_rows // n_workers
    n_chunks = per_w // _CHUNK
    n_outer = n_chunks // _NBUF
    l_per_w = n_l // _LW
    b_per_w = n_batch // _BW
    nc = plsc.get_sparse_core_info().num_cores

    def body(idx_hbm, tf_hbm, vt_hbm, tt_hbm, out_hbm, *refs):
        idx_v, tf_v, tt_v = refs[0], refs[1], refs[2]
        rows = refs[3:3 + _NBUF]
        gsem = refs[3 + _NBUF:3 + 2 * _NBUF]
        wsem = refs[3 + 2 * _NBUF:3 + 3 * _NBUF]

        wid = lax.axis_index("s") * nc + lax.axis_index("c")
        base = wid * per_w
        lh = wid // _BW
        bb = wid % _BW

        pltpu.sync_copy(idx_hbm.at[pl.ds(base, per_w)], idx_v)
        pltpu.sync_copy(tf_hbm.at[pl.ds(base, per_w)], tf_v)
        pltpu.sync_copy(tt_hbm, tt_v)
        t0 = [tt_v[0, pl.ds(16 * k, 16)] for k in range(_NVREG)]
        dt = [tt_v[1, pl.ds(16 * k, 16)] - t0[k] for k in range(_NVREG)]

        def out_off(c):
            l = lh * l_per_w + c // 2
            b0 = bb * b_per_w + (c % 2) * _CHUNK
            return l * n_batch + b0

        def gather_args(c, b):
            return (vt_hbm.at[idx_v.at[pl.ds(c * _CHUNK, _CHUNK)]], rows[b],
                    gsem[b])

        def writeback_args(c, b):
            return (rows[b], out_hbm.at[pl.ds(out_off(c), _CHUNK)], wsem[b])

        def gather(c, b):
            pltpu.async_copy(*gather_args(c, b))

        def gather_wait(c, b):
            pltpu.make_async_copy(*gather_args(c, b)).wait()

        def writeback(c, b):
            pltpu.async_copy(*writeback_args(c, b))

        def writeback_wait(c, b):
            pltpu.make_async_copy(*writeback_args(c, b)).wait()

        for c in range(_NBUF - 2):
            gather(c, c)

        def compute(buf, c):
            def group_body(g, carry):
                tvec = tf_v[pl.ds(c * _CHUNK + g * _GROUP, _GROUP)]
                for j in range(_GROUP):
                    tsp = tvec.at[jnp.full((16,), j, jnp.int32)].get(
                        mode="promise_in_bounds")
                    r = g * _GROUP + j
                    for k in range(_NVREG):
                        sl = pl.ds(16 * k, 16)
                        buf[r, sl] = buf[r, sl] + (t0[k] + tsp * dt[k])
                return carry

            lax.fori_loop(0, _CHUNK // _GROUP, group_body, 0)

        def outer_body(gi, carry):
            for b in range(_NBUF):
                c = gi * _NBUF + b
                gather_wait(c, b)
                compute(rows[b], c)
                writeback(c, b)
                br = (b + _NBUF - 2) % _NBUF

                @pl.when(c >= 2)
                def _():
                    writeback_wait(c - 2, br)

                @pl.when(c <= n_chunks - 1 - (_NBUF - 2))
                def _():
                    gather(c + _NBUF - 2, br)

            return carry

        lax.fori_loop(0, n_outer, outer_body, 0)
        writeback_wait(n_chunks - 2, (n_chunks - 2) % _NBUF)
        writeback_wait(n_chunks - 1, (n_chunks - 1) % _NBUF)

    return pl.kernel(
        body,
        out_type=jax.ShapeDtypeStruct((n_rows, _HIDDEN), jnp.float32),
        mesh=plsc.VectorSubcoreMesh(core_axis_name="c", subcore_axis_name="s"),
        compiler_params=pltpu.CompilerParams(needs_layout_passes=False),
        scratch_types=(
            [
                pltpu.VMEM((per_w,), jnp.int32),
                pltpu.VMEM((per_w,), jnp.float32),
                pltpu.VMEM((2, _HIDDEN), jnp.float32),
            ]
            + [pltpu.VMEM((_CHUNK, _HIDDEN), jnp.float32)] * _NBUF
            + [pltpu.SemaphoreType.DMA] * (2 * _NBUF)
        ),
    )(idx, tf, vocab_table, type_table)


def _worker_order(x, b, l):
    lpw, bpw = l // _LW, b // _BW
    xt = x.T.reshape(_LW, lpw, _BW, bpw // _CHUNK, _CHUNK)
    return xt.transpose(0, 2, 1, 3, 4).reshape(-1)


def kernel(vocab, type, vocab_table, type_table):
    b, l = vocab.shape
    info = plsc.get_sparse_core_info()
    n_workers = info.num_cores * info.num_subcores
    idx = _worker_order(vocab, b, l)
    tf = _worker_order(type.astype(jnp.float32), b, l)
    out = _embed(idx, tf, vocab_table, type_table, b, l, n_workers)
    return out.reshape(l, b, _HIDDEN).transpose(1, 0, 2)
